# baseline (speedup 1.0000x reference)
import functools

import jax
import jax.numpy as jnp
from jax import lax
from jax.experimental import pallas as pl
from jax.experimental.pallas import tpu as pltpu
from jax.experimental.pallas import tpu_sc as plsc

_NC = 2
_NS = 16
_NW = _NC * _NS
_IDX_W = 128


def _make_pool(V, Dp, B, LIN):
    items_per_w = B // _NW
    flat_per_w = items_per_w * LIN
    n_groups = items_per_w // _IDX_W

    mesh = plsc.VectorSubcoreMesh(core_axis_name="c", subcore_axis_name="s")

    @functools.partial(
        pl.kernel,
        out_type=jax.ShapeDtypeStruct((B, Dp), jnp.float32),
        mesh=mesh,
        compiler_params=pltpu.CompilerParams(use_tc_tiling_on_sc=False),
        scratch_types=[
            pltpu.VMEM((flat_per_w,), jnp.int32),
        ]
        + [pltpu.VMEM((_IDX_W, Dp), jnp.float32)] * 4
        + [pltpu.SemaphoreType.DMA] * 4,
    )
    def pool(table_h, idxf_h, out_h,
             idx_v, ov0, ov1, ov2, ov3, sm0, sm1, sm2, sm3):
        wid = lax.axis_index("s") * _NC + lax.axis_index("c")
        outvs = (ov0, ov1, ov2, ov3)
        sems = (sm0, sm1, sm2, sm3)

        for j in range(LIN):
            pltpu.sync_copy(
                idxf_h.at[pl.ds(j * B + wid * items_per_w, items_per_w)],
                idx_v.at[pl.ds(j * items_per_w, items_per_w)])

        def start_gather(g, j, add):
            return pltpu.async_copy(
                table_h.at[idx_v.at[pl.ds(j * items_per_w + g * _IDX_W,
                                          _IDX_W)]],
                outvs[g], sems[g], add=add)

        hs = [start_gather(g, 0, False) for g in range(n_groups)]
        for j in range(1, LIN):
            for g in range(n_groups):
                hs[g].wait()
                hs[g] = start_gather(g, j, True)
        for g in range(n_groups):
            hs[g].wait()
            hs[g] = pltpu.async_copy(
                outvs[g],
                out_h.at[pl.ds(wid * items_per_w + g * _IDX_W, _IDX_W)],
                sems[g])
        for g in range(n_groups):
            hs[g].wait()

    return pool


def _pad_body(x_ref, o_ref):
    D = x_ref.shape[1]
    o_ref[:, :D] = x_ref[...]
    o_ref[:, D:] = jnp.zeros_like(o_ref[:, D:])


def _make_pad(V, D, Dp):
    blk = 10000
    return pl.pallas_call(
        _pad_body,
        grid=(V // blk,),
        in_specs=[pl.BlockSpec((blk, D), lambda i: (i, 0))],
        out_specs=pl.BlockSpec((blk, Dp), lambda i: (i, 0)),
        out_shape=jax.ShapeDtypeStruct((V, Dp), jnp.float32),
    )


def _mlp_body(x_ref, w1_ref, b1_ref, w2_ref, b2_ref, w3_ref, b3_ref, o_ref):
    x = x_ref[...]
    h = jnp.dot(x, w1_ref[...], preferred_element_type=jnp.float32) + b1_ref[...]
    h = jnp.maximum(h, 0.0)
    h = jnp.dot(h, w2_ref[...], preferred_element_type=jnp.float32) + b2_ref[...]
    h = jnp.maximum(h, 0.0)
    o_ref[...] = jnp.sum(h * w3_ref[...], axis=1) + b3_ref[0]


def _make_mlp(B, Dp, H):
    blk = 4096
    grid = (B // blk,)
    return pl.pallas_call(
        _mlp_body,
        grid=grid,
        in_specs=[
            pl.BlockSpec((blk, Dp), lambda i: (i, 0)),
            pl.BlockSpec((Dp, H), lambda i: (0, 0)),
            pl.BlockSpec((1, H), lambda i: (0, 0)),
            pl.BlockSpec((H, H), lambda i: (0, 0)),
            pl.BlockSpec((1, H), lambda i: (0, 0)),
            pl.BlockSpec((1, H), lambda i: (0, 0)),
            pl.BlockSpec(memory_space=pltpu.SMEM),
        ],
        out_specs=pl.BlockSpec((blk,), lambda i: (i,)),
        out_shape=jax.ShapeDtypeStruct((B,), jnp.float32),
    )


def kernel(player_indices, table, W1, b1, W2, b2, W3, b3):
    B, LIN = player_indices.shape
    V, D = table.shape
    H = W1.shape[1]
    Dp = 128

    table_p = _make_pad(V, D, Dp)(table)

    idx_flat = player_indices.astype(jnp.int32).T.reshape(LIN * B)
    pooled = _make_pool(V, Dp, B, LIN)(table_p, idx_flat)

    W1p = jnp.pad(W1 * (1.0 / LIN), ((0, Dp - D), (0, 0)))
    out = _make_mlp(B, Dp, H)(
        pooled, W1p, b1.reshape(1, H), W2, b2.reshape(1, H),
        W3.reshape(1, H), b3)
    return out


# speedup vs baseline: 1.0556x; 1.0556x over previous
import functools

import jax
import jax.numpy as jnp
from jax import lax
from jax.experimental import pallas as pl
from jax.experimental.pallas import tpu as pltpu
from jax.experimental.pallas import tpu_sc as plsc

_NC = 2
_NS = 16
_NW = _NC * _NS
_IDX_W = 128


def _make_pool(V, Dp, B, LIN):
    items_per_w = B // _NW
    flat_per_w = items_per_w * LIN
    n_groups = items_per_w // _IDX_W

    mesh = plsc.VectorSubcoreMesh(core_axis_name="c", subcore_axis_name="s")

    @functools.partial(
        pl.kernel,
        out_type=jax.ShapeDtypeStruct((B, Dp), jnp.float32),
        mesh=mesh,
        compiler_params=pltpu.CompilerParams(use_tc_tiling_on_sc=False),
        scratch_types=[
            pltpu.VMEM((flat_per_w,), jnp.int32),
        ]
        + [pltpu.VMEM((_IDX_W, Dp), jnp.float32)] * 4
        + [pltpu.SemaphoreType.DMA] * 4,
    )
    def pool(table_h, idxf_h, out_h,
             idx_v, ov0, ov1, ov2, ov3, sm0, sm1, sm2, sm3):
        wid = lax.axis_index("s") * _NC + lax.axis_index("c")
        outvs = (ov0, ov1, ov2, ov3)
        sems = (sm0, sm1, sm2, sm3)

        for j in range(LIN):
            pltpu.sync_copy(
                idxf_h.at[pl.ds(j * B + wid * items_per_w, items_per_w)],
                idx_v.at[pl.ds(j * items_per_w, items_per_w)])

        def start_gather(g, j, add):
            return pltpu.async_copy(
                table_h.at[idx_v.at[pl.ds(j * items_per_w + g * _IDX_W,
                                          _IDX_W)]],
                outvs[g], sems[g], add=add)

        hs = [start_gather(g, 0, False) for g in range(n_groups)]
        for j in range(1, LIN):
            for g in range(n_groups):
                hs[g].wait()
                hs[g] = start_gather(g, j, True)
        for g in range(n_groups):
            hs[g].wait()
            hs[g] = pltpu.async_copy(
                outvs[g],
                out_h.at[pl.ds(wid * items_per_w + g * _IDX_W, _IDX_W)],
                sems[g])
        for g in range(n_groups):
            hs[g].wait()

    return pool


def _pad_body(x_ref, o_ref):
    D = x_ref.shape[1]
    o_ref[:, :D] = x_ref[...]
    o_ref[:, D:] = jnp.zeros_like(o_ref[:, D:])


def _make_pad(V, D, Dp):
    blk = 10000
    return pl.pallas_call(
        _pad_body,
        grid=(V // blk,),
        in_specs=[pl.BlockSpec((blk, D), lambda i: (i, 0))],
        out_specs=pl.BlockSpec((blk, Dp), lambda i: (i, 0)),
        out_shape=jax.ShapeDtypeStruct((V, Dp), jnp.float32),
    )


def _mlp_body(x_ref, w1_ref, b1_ref, w2_ref, b2_ref, w3m_ref, b3_ref, o_ref):
    x = x_ref[...]
    h = jnp.dot(x, w1_ref[...], preferred_element_type=jnp.float32) + b1_ref[...]
    h = jnp.maximum(h, 0.0)
    h = jnp.dot(h, w2_ref[...], preferred_element_type=jnp.float32) + b2_ref[...]
    h = jnp.maximum(h, 0.0)
    r = jnp.dot(h, w3m_ref[...], preferred_element_type=jnp.float32)
    o_ref[...] = r[:, 0] + b3_ref[0]


def _make_mlp(B, Dp, H):
    blk = 4096
    grid = (B // blk,)
    return pl.pallas_call(
        _mlp_body,
        grid=grid,
        in_specs=[
            pl.BlockSpec((blk, Dp), lambda i: (i, 0)),
            pl.BlockSpec((Dp, H), lambda i: (0, 0)),
            pl.BlockSpec((1, H), lambda i: (0, 0)),
            pl.BlockSpec((H, H), lambda i: (0, 0)),
            pl.BlockSpec((1, H), lambda i: (0, 0)),
            pl.BlockSpec((H, H), lambda i: (0, 0)),
            pl.BlockSpec(memory_space=pltpu.SMEM),
        ],
        out_specs=pl.BlockSpec((blk,), lambda i: (i,)),
        out_shape=jax.ShapeDtypeStruct((B,), jnp.float32),
    )


def kernel(player_indices, table, W1, b1, W2, b2, W3, b3):
    B, LIN = player_indices.shape
    V, D = table.shape
    H = W1.shape[1]
    Dp = 128

    table_p = _make_pad(V, D, Dp)(table)

    idx_flat = player_indices.astype(jnp.int32).T.reshape(LIN * B)
    pooled = _make_pool(V, Dp, B, LIN)(table_p, idx_flat)

    W1p = jnp.pad(W1 * (1.0 / LIN), ((0, Dp - D), (0, 0)))
    W3m = jnp.pad(W3.reshape(H, 1), ((0, 0), (0, H - 1)))
    out = _make_mlp(B, Dp, H)(
        pooled, W1p, b1.reshape(1, H), W2, b2.reshape(1, H),
        W3m, b3)
    return out
